# Initial kernel scaffold; baseline (speedup 1.0000x reference)
import jax
import jax.numpy as jnp
from jax.experimental import pallas as pl


def kernel(embeddings, span_positions, W, att_src, att_dst, bias):
    raise NotImplementedError("write your pallas kernel here")


# speedup vs baseline: 3.1648x; 3.1648x over previous
import functools

import jax
import jax.numpy as jnp
from jax.experimental import pallas as pl

_EMB_DIM = 384
_HEADS = 4
_LAMBDA = 5.0
_THRESH = 0.1
_SLOPE = 0.2

_BLK = 256
_HALO = 16
_EXT = _BLK + 2 * _HALO


def _gat_band_kernel(emb_ref, w_ref, asrc_ref, adst_ref, bias_ref, out_ref):
    i = pl.program_id(0)

    emb_ext = emb_ref[pl.ds(i * _BLK, _EXT), :]
    norms = jnp.sqrt(jnp.sum(emb_ext * emb_ext, axis=1, keepdims=True))
    en_ext = emb_ext / jnp.maximum(norms, 1e-12)
    en_blk = en_ext[_HALO:_HALO + _BLK, :]

    sim = jax.lax.dot_general(
        en_blk, en_ext, (((1,), (1,)), ((), ())),
        preferred_element_type=jnp.float32)

    rows = jax.lax.broadcasted_iota(jnp.int32, (_BLK, _EXT), 0)
    cols = jax.lax.broadcasted_iota(jnp.int32, (_BLK, _EXT), 1)
    dist = jnp.abs(rows + _HALO - cols).astype(jnp.float32)
    adj = sim * jnp.exp(-dist / _LAMBDA)
    mask = adj > _THRESH

    x_ext = jax.lax.dot_general(
        emb_ext, w_ref[...], (((1,), (0,)), ((), ())),
        preferred_element_type=jnp.float32)

    acc = jnp.zeros((_BLK, _EMB_DIM), dtype=jnp.float32)
    for h in range(_HEADS):
        xh = x_ext[:, h * _EMB_DIM:(h + 1) * _EMB_DIM]
        xh_blk = xh[_HALO:_HALO + _BLK, :]
        a_src = jax.lax.dot_general(
            asrc_ref[h:h + 1, :], xh, (((1,), (1,)), ((), ())),
            preferred_element_type=jnp.float32)
        a_dst = jax.lax.dot_general(
            xh_blk, adst_ref[h:h + 1, :], (((1,), (1,)), ((), ())),
            preferred_element_type=jnp.float32)
        logits = a_dst + a_src
        logits = jnp.where(logits >= 0, logits, _SLOPE * logits)
        logits = jnp.where(mask, logits, -1e30)
        m = jnp.max(logits, axis=1, keepdims=True)
        p = jnp.exp(logits - m)
        p = jnp.where(mask, p, 0.0)
        denom = jnp.sum(p, axis=1, keepdims=True)
        alpha = p / denom
        acc = acc + jax.lax.dot_general(
            alpha, xh, (((1,), (0,)), ((), ())),
            preferred_element_type=jnp.float32)

    out_ref[...] = acc * (1.0 / _HEADS) + bias_ref[...][None, :]


@functools.partial(jax.jit, static_argnames=())
def kernel(embeddings, span_positions, W, att_src, att_dst, bias):
    del span_positions
    n, d = embeddings.shape
    grid = (n // _BLK,)
    emb_p = jnp.pad(embeddings, ((_HALO, _HALO), (0, 0)))
    out = pl.pallas_call(
        _gat_band_kernel,
        grid=grid,
        in_specs=[
            pl.BlockSpec((n + 2 * _HALO, d), lambda i: (0, 0)),
            pl.BlockSpec(W.shape, lambda i: (0, 0)),
            pl.BlockSpec(att_src.shape, lambda i: (0, 0)),
            pl.BlockSpec(att_dst.shape, lambda i: (0, 0)),
            pl.BlockSpec(bias.shape, lambda i: (0,)),
        ],
        out_specs=pl.BlockSpec((_BLK, d), lambda i: (i, 0)),
        out_shape=jax.ShapeDtypeStruct((n, d), jnp.float32),
    )(emb_p, W, att_src, att_dst, bias)
    return out
